# baseline (speedup 1.0000x reference)
import jax
import jax.numpy as jnp
from jax.experimental import pallas as pl
from jax.experimental.pallas import tpu as pltpu

N = 2048
D = 1024
E = 8
TOPK = 2
DFF = 4096
BT = 256
FBLK = 1024
NF = DFF // FBLK
NA = N * TOPK
NAP = NA + 512
CHK = 128
NCK = N // CHK


def _gate_kernel(x_ref, gw_ref, gb_ref, scores_ref, dest_ref, wt_ref, cs_ref):
    x = x_ref[...]
    gw = gw_ref[...]
    logits = jax.lax.dot_general(
        x, gw, (((1,), (1,)), ((), ())),
        preferred_element_type=jnp.float32) + gb_ref[...]
    scores = jax.nn.sigmoid(logits)
    scores_ref[...] = scores
    col = jax.lax.broadcasted_iota(jnp.int32, scores.shape, 1)
    m1 = jnp.max(scores, axis=1, keepdims=True)
    a1 = jnp.min(jnp.where(scores == m1, col, E), axis=1, keepdims=True)
    masked = jnp.where(col == a1, -1.0, scores)
    m2 = jnp.max(masked, axis=1, keepdims=True)
    a2 = jnp.min(jnp.where(masked == m2, col, E), axis=1, keepdims=True)

    oh = ((col == a1) | (col == a2)).astype(jnp.float32)
    ohc = oh.reshape(NCK, CHK, E)
    li = jax.lax.broadcasted_iota(jnp.int32, (CHK, CHK), 0)
    lj = jax.lax.broadcasted_iota(jnp.int32, (CHK, CHK), 1)
    lx = (lj < li).astype(jnp.float32)
    lxb = jnp.broadcast_to(lx, (NCK, CHK, CHK))
    chunk_pref = jax.lax.dot_general(
        lxb, ohc, (((2,), (1,)), ((0,), (0,))),
        precision=jax.lax.Precision.HIGHEST,
        preferred_element_type=jnp.float32)
    chunk_sums = jnp.sum(ohc, axis=1)
    ci = jax.lax.broadcasted_iota(jnp.int32, (NCK, NCK), 0)
    cj = jax.lax.broadcasted_iota(jnp.int32, (NCK, NCK), 1)
    cx = (cj < ci).astype(jnp.float32)
    chunk_base = jax.lax.dot_general(
        cx, chunk_sums, (((1,), (0,)), ((), ())),
        precision=jax.lax.Precision.HIGHEST,
        preferred_element_type=jnp.float32)
    prefix = (chunk_pref + chunk_base[:, None, :]).reshape(N, E)

    counts = jnp.sum(chunk_sums, axis=0, keepdims=True)
    ei = jax.lax.broadcasted_iota(jnp.int32, (E, E), 0)
    ej = jax.lax.broadcasted_iota(jnp.int32, (E, E), 1)
    ex = (ei < ej).astype(jnp.float32)
    starts = jax.lax.dot_general(
        counts, ex, (((1,), (0,)), ((), ())),
        precision=jax.lax.Precision.HIGHEST,
        preferred_element_type=jnp.float32)

    dest_e = starts + prefix
    d1 = jnp.sum(jnp.where(col == a1, dest_e, 0.0), axis=1, keepdims=True)
    d2 = jnp.sum(jnp.where(col == a2, dest_e, 0.0), axis=1, keepdims=True)
    zf = jnp.zeros((x.shape[0], E - TOPK), dtype=jnp.float32)
    dest_ref[...] = (jnp.concatenate(
        [d1, d2, zf], axis=1) + 0.5).astype(jnp.int32)
    wt_ref[...] = jnp.concatenate([m1, m2, zf], axis=1)
    cs_ref[...] = (jnp.concatenate(
        [counts, starts], axis=0) + 0.5).astype(jnp.int32)


def _moe_kernel(counts_ref, starts_ref, tok_ref,
                x_ref, w1_ref, b1_ref, w2_ref, b2_ref, wgt_ref,
                out_ref, xg_ref, acc_ref):
    e = pl.program_id(0)
    f = pl.program_id(1)
    cnt = counts_ref[e]
    start = starts_ref[e]
    nblk = (cnt + BT - 1) // BT

    @pl.when(jnp.logical_and(e == 0, f == 0))
    def _init():
        out_ref[...] = jnp.zeros_like(out_ref)

    def block_body(b, _):
        off = start + b * BT
        nv = jnp.clip(cnt - b * BT, 0, BT)
        row = pl.ds(b * BT, BT)

        @pl.when(f == 0)
        def _gather():
            def gbody(r, _):
                t = tok_ref[off + r]
                xg_ref[pl.ds(b * BT + r, 1), :] = x_ref[pl.ds(t, 1), :]
                return 0
            jax.lax.fori_loop(0, BT, gbody, 0, unroll=True)

        xs = xg_ref[row, :]
        h = jax.lax.dot_general(
            xs, w1_ref[0], (((1,), (1,)), ((), ())),
            preferred_element_type=jnp.float32) + b1_ref[0]
        h = jax.nn.gelu(h, approximate=True)
        part = jax.lax.dot_general(
            h, w2_ref[0], (((1,), (1,)), ((), ())),
            preferred_element_type=jnp.float32)

        @pl.when(f == 0)
        def _first():
            acc_ref[row, :] = part + b2_ref[0]

        @pl.when(f > 0)
        def _rest():
            acc_ref[row, :] += part

        @pl.when(f == NF - 1)
        def _scatter():
            ridx = jax.lax.broadcasted_iota(jnp.int32, (BT, 1), 0)
            w = jnp.where(ridx < nv, wgt_ref[pl.ds(off, BT), :], 0.0)
            acc_ref[row, :] *= w
            def sbody(r, _):
                t = tok_ref[off + r]
                out_ref[pl.ds(t, 1), :] += acc_ref[pl.ds(b * BT + r, 1), :]
                return 0
            jax.lax.fori_loop(0, BT, sbody, 0, unroll=True)
        return 0

    jax.lax.fori_loop(0, nblk, block_body, 0)


def kernel(x, gate_w, gate_b, w1, b1, w2, b2):
    bsz, t, d = x.shape
    x_flat = x.reshape(N, D)

    scores, dest8, wt8, cs = pl.pallas_call(
        _gate_kernel,
        out_shape=(
            jax.ShapeDtypeStruct((N, E), jnp.float32),
            jax.ShapeDtypeStruct((N, E), jnp.int32),
            jax.ShapeDtypeStruct((N, E), jnp.float32),
            jax.ShapeDtypeStruct((2, E), jnp.int32),
        ),
    )(x_flat, gate_w, gate_b)

    dest = dest8[:, :TOPK].reshape(-1)
    wa = wt8[:, :TOPK].reshape(-1)
    ta = (jnp.arange(NA, dtype=jnp.int32) // TOPK).astype(jnp.float32)
    upd = jnp.stack([ta, wa], axis=-1)
    grouped = jnp.zeros((NAP, 2), jnp.float32).at[dest].set(
        upd, unique_indices=True, mode="promise_in_bounds")
    tok_pad = grouped[:, 0].astype(jnp.int32)
    wgt_pad = grouped[:, 1:2]
    counts, starts = cs[0], cs[1]

    b1r = b1.reshape(E, 1, DFF)
    b2r = b2.reshape(E, 1, D)

    grid_spec = pltpu.PrefetchScalarGridSpec(
        num_scalar_prefetch=3,
        grid=(E, NF),
        in_specs=[
            pl.BlockSpec((N, D), lambda e, f, *s: (0, 0)),
            pl.BlockSpec((1, FBLK, D), lambda e, f, *s: (e, f, 0)),
            pl.BlockSpec((1, 1, FBLK), lambda e, f, *s: (e, 0, f)),
            pl.BlockSpec((1, D, FBLK), lambda e, f, *s: (e, 0, f)),
            pl.BlockSpec((1, 1, D), lambda e, f, *s: (e, 0, 0)),
            pl.BlockSpec((NAP, 1), lambda e, f, *s: (0, 0)),
        ],
        out_specs=pl.BlockSpec((N, D), lambda e, f, *s: (0, 0)),
        scratch_shapes=[
            pltpu.VMEM((N, D), jnp.float32),
            pltpu.VMEM((N, D), jnp.float32),
        ],
    )

    out = pl.pallas_call(
        _moe_kernel,
        grid_spec=grid_spec,
        out_shape=jax.ShapeDtypeStruct((N, D), jnp.float32),
        compiler_params=pltpu.CompilerParams(
            dimension_semantics=("arbitrary", "arbitrary"),
        ),
    )(counts, starts, tok_pad, x_flat, w1, b1r, w2, b2r, wgt_pad)

    return (out.reshape(bsz, t, d), scores.reshape(bsz, t, E))


# speedup vs baseline: 1.0442x; 1.0442x over previous
import jax
import jax.numpy as jnp
from jax.experimental import pallas as pl
from jax.experimental.pallas import tpu as pltpu

N = 2048
D = 1024
E = 8
TOPK = 2
DFF = 4096
BT = 256
FBLK = 1024
NF = DFF // FBLK
NA = N * TOPK
NAP = NA + 512
CHK = 128
NCK = N // CHK


def _gate_kernel(x_ref, gw_ref, gb_ref, scores_ref, dest_ref, wt_ref, cs_ref):
    x = x_ref[...]
    gw = gw_ref[...]
    logits = jax.lax.dot_general(
        x, gw, (((1,), (1,)), ((), ())),
        preferred_element_type=jnp.float32) + gb_ref[...]
    scores = jax.nn.sigmoid(logits)
    scores_ref[...] = scores
    col = jax.lax.broadcasted_iota(jnp.int32, scores.shape, 1)
    m1 = jnp.max(scores, axis=1, keepdims=True)
    a1 = jnp.min(jnp.where(scores == m1, col, E), axis=1, keepdims=True)
    masked = jnp.where(col == a1, -1.0, scores)
    m2 = jnp.max(masked, axis=1, keepdims=True)
    a2 = jnp.min(jnp.where(masked == m2, col, E), axis=1, keepdims=True)

    oh = ((col == a1) | (col == a2)).astype(jnp.float32)
    counts = jnp.sum(oh, axis=0, keepdims=True)
    ei = jax.lax.broadcasted_iota(jnp.int32, (E, E), 0)
    ej = jax.lax.broadcasted_iota(jnp.int32, (E, E), 1)
    ex = (ei < ej).astype(jnp.float32)
    starts = jax.lax.dot_general(
        counts, ex, (((1,), (0,)), ((), ())),
        precision=jax.lax.Precision.HIGHEST,
        preferred_element_type=jnp.float32)
    z = jnp.zeros((x.shape[0], E - TOPK), dtype=jnp.int32)
    dest_ref[...] = jnp.concatenate([a1, a2, z], axis=1)
    wt_ref[...] = jnp.concatenate([m1, m2, z.astype(jnp.float32)], axis=1)
    cs_ref[...] = (jnp.concatenate(
        [counts, starts], axis=0) + 0.5).astype(jnp.int32)


def _moe_kernel(counts_ref, starts_ref, tok_ref,
                x_ref, w1_ref, b1_ref, w2_ref, b2_ref, wgt_ref,
                out_ref, xg_ref, acc_ref):
    e = pl.program_id(0)
    f = pl.program_id(1)
    cnt = counts_ref[e]
    start = starts_ref[e]
    nblk = (cnt + BT - 1) // BT

    @pl.when(jnp.logical_and(e == 0, f == 0))
    def _init():
        out_ref[...] = jnp.zeros_like(out_ref)

    def block_body(b, _):
        off = start + b * BT
        nv = jnp.clip(cnt - b * BT, 0, BT)
        row = pl.ds(b * BT, BT)

        @pl.when(f == 0)
        def _gather():
            def gbody(r, _):
                t = tok_ref[off + r]
                xg_ref[pl.ds(b * BT + r, 1), :] = x_ref[pl.ds(t, 1), :]
                return 0
            jax.lax.fori_loop(0, BT, gbody, 0, unroll=True)

        xs = xg_ref[row, :]
        h = jax.lax.dot_general(
            xs, w1_ref[0], (((1,), (1,)), ((), ())),
            preferred_element_type=jnp.float32) + b1_ref[0]
        h = jax.nn.gelu(h, approximate=True)
        part = jax.lax.dot_general(
            h, w2_ref[0], (((1,), (1,)), ((), ())),
            preferred_element_type=jnp.float32)

        @pl.when(f == 0)
        def _first():
            acc_ref[row, :] = part + b2_ref[0]

        @pl.when(f > 0)
        def _rest():
            acc_ref[row, :] += part

        @pl.when(f == NF - 1)
        def _scatter():
            ridx = jax.lax.broadcasted_iota(jnp.int32, (BT, 1), 0)
            w = jnp.where(ridx < nv, wgt_ref[pl.ds(off, BT), :], 0.0)
            acc_ref[row, :] *= w
            def sbody(r, _):
                t = tok_ref[off + r]
                out_ref[pl.ds(t, 1), :] += acc_ref[pl.ds(b * BT + r, 1), :]
                return 0
            jax.lax.fori_loop(0, BT, sbody, 0, unroll=True)
        return 0

    jax.lax.fori_loop(0, nblk, block_body, 0)


def kernel(x, gate_w, gate_b, w1, b1, w2, b2):
    bsz, t, d = x.shape
    x_flat = x.reshape(N, D)

    scores, dest8, wt8, cs = pl.pallas_call(
        _gate_kernel,
        out_shape=(
            jax.ShapeDtypeStruct((N, E), jnp.float32),
            jax.ShapeDtypeStruct((N, E), jnp.int32),
            jax.ShapeDtypeStruct((N, E), jnp.float32),
            jax.ShapeDtypeStruct((2, E), jnp.int32),
        ),
    )(x_flat, gate_w, gate_b)

    ea = dest8[:, :TOPK].reshape(-1)
    wa = wt8[:, :TOPK].reshape(-1)
    ta = (jnp.arange(NA, dtype=jnp.int32) // TOPK)
    _, sorted_tok, sorted_w = jax.lax.sort(
        (ea, ta, wa), dimension=0, is_stable=True, num_keys=1)
    tok_pad = jnp.concatenate(
        [sorted_tok, jnp.zeros((NAP - NA,), jnp.int32)]).astype(jnp.int32)
    wgt_pad = jnp.concatenate(
        [sorted_w, jnp.zeros((NAP - NA,), jnp.float32)]).reshape(NAP, 1)
    counts, starts = cs[0], cs[1]

    b1r = b1.reshape(E, 1, DFF)
    b2r = b2.reshape(E, 1, D)

    grid_spec = pltpu.PrefetchScalarGridSpec(
        num_scalar_prefetch=3,
        grid=(E, NF),
        in_specs=[
            pl.BlockSpec((N, D), lambda e, f, *s: (0, 0)),
            pl.BlockSpec((1, FBLK, D), lambda e, f, *s: (e, f, 0)),
            pl.BlockSpec((1, 1, FBLK), lambda e, f, *s: (e, 0, f)),
            pl.BlockSpec((1, D, FBLK), lambda e, f, *s: (e, 0, f)),
            pl.BlockSpec((1, 1, D), lambda e, f, *s: (e, 0, 0)),
            pl.BlockSpec((NAP, 1), lambda e, f, *s: (0, 0)),
        ],
        out_specs=pl.BlockSpec((N, D), lambda e, f, *s: (0, 0)),
        scratch_shapes=[
            pltpu.VMEM((N, D), jnp.float32),
            pltpu.VMEM((N, D), jnp.float32),
        ],
    )

    out = pl.pallas_call(
        _moe_kernel,
        grid_spec=grid_spec,
        out_shape=jax.ShapeDtypeStruct((N, D), jnp.float32),
        compiler_params=pltpu.CompilerParams(
            dimension_semantics=("arbitrary", "arbitrary"),
        ),
    )(counts, starts, tok_pad, x_flat, w1, b1r, w2, b2r, wgt_pad)

    return (out.reshape(bsz, t, d), scores.reshape(bsz, t, E))
